# Initial kernel scaffold; baseline (speedup 1.0000x reference)
import jax
import jax.numpy as jnp
from jax.experimental import pallas as pl


def kernel(x, Z, shift):
    raise NotImplementedError("write your pallas kernel here")


# speedup vs baseline: 22.4887x; 22.4887x over previous
import functools

import jax
import jax.numpy as jnp
from jax import lax
from jax.experimental import pallas as pl
from jax.experimental.pallas import tpu as pltpu
from jax.experimental.pallas import tpu_sc as plsc

_N_ATOMS = 100000
_NUM_WORKERS = 32
_PER_W = 3136
_N_PAD = _NUM_WORKERS * _PER_W
_TABLE_PAD = 128
_LANES = 16


def _sc_body(x_hbm, z_hbm, tab_hbm, out_hbm, tab_v, x_v, z_v, o_v):
    wid = lax.axis_index("s") * 2 + lax.axis_index("c")
    base = wid * _PER_W
    pltpu.sync_copy(tab_hbm, tab_v)
    pltpu.sync_copy(x_hbm.at[pl.ds(base, _PER_W)], x_v)
    pltpu.sync_copy(z_hbm.at[pl.ds(base, _PER_W)], z_v)

    def step(i, carry):
        off = i * _LANES
        z = z_v[pl.ds(off, _LANES)]
        s = plsc.load_gather(tab_v, [z])
        o_v[pl.ds(off, _LANES)] = x_v[pl.ds(off, _LANES)] - s
        return carry

    lax.fori_loop(0, _PER_W // _LANES, step, 0)
    pltpu.sync_copy(o_v, out_hbm.at[pl.ds(base, _PER_W)])


_sc_call = functools.partial(
    pl.kernel,
    out_type=jax.ShapeDtypeStruct((_N_PAD,), jnp.float32),
    mesh=plsc.VectorSubcoreMesh(core_axis_name="c", subcore_axis_name="s"),
    compiler_params=pltpu.CompilerParams(needs_layout_passes=False),
    scratch_types=[
        pltpu.VMEM((_TABLE_PAD,), jnp.float32),
        pltpu.VMEM((_PER_W,), jnp.float32),
        pltpu.VMEM((_PER_W,), jnp.int32),
        pltpu.VMEM((_PER_W,), jnp.float32),
    ],
)(_sc_body)


@jax.jit
def kernel(x, Z, shift):
    n = x.shape[0]
    xp = jnp.pad(x, (0, _N_PAD - n))
    zp = jnp.pad(Z, (0, _N_PAD - n))
    tab = jnp.pad(shift[:, 0], (0, _TABLE_PAD - shift.shape[0]))
    out = _sc_call(xp, zp, tab)
    return out[:n]


# speedup vs baseline: 25.2857x; 1.1244x over previous
import functools

import jax
import jax.numpy as jnp
from jax import lax
from jax.experimental import pallas as pl
from jax.experimental.pallas import tpu as pltpu
from jax.experimental.pallas import tpu_sc as plsc

_N_ATOMS = 100000
_NUM_WORKERS = 32
_PER_W = 3136
_LAST_W = _N_ATOMS - 31 * _PER_W
_N_SPECIES = 120
_LANES = 16


def _sc_body(x_hbm, z_hbm, tab_hbm, out_hbm, tab_v, x_v, z_v, o_v, sem):
    wid = lax.axis_index("s") * 2 + lax.axis_index("c")
    base = wid * _PER_W

    def work(size, unroll):
        cp_t = pltpu.make_async_copy(tab_hbm, tab_v, sem)
        cp_x = pltpu.make_async_copy(
            x_hbm.at[pl.ds(base, size)], x_v.at[pl.ds(0, size)], sem)
        cp_z = pltpu.make_async_copy(
            z_hbm.at[pl.ds(base, size)], z_v.at[pl.ds(0, size)], sem)
        cp_t.start()
        cp_x.start()
        cp_z.start()
        cp_t.wait()
        cp_x.wait()
        cp_z.wait()

        step = unroll * _LANES

        def body(i, carry):
            off = i * step
            for j in range(unroll):
                o = off + j * _LANES
                z = z_v[pl.ds(o, _LANES)]
                s = plsc.load_gather(tab_v, [z])
                o_v[pl.ds(o, _LANES)] = x_v[pl.ds(o, _LANES)] - s
            return carry

        lax.fori_loop(0, size // step, body, 0)
        pltpu.sync_copy(o_v.at[pl.ds(0, size)], out_hbm.at[pl.ds(base, size)])

    @pl.when(wid < _NUM_WORKERS - 1)
    def _():
        work(_PER_W, 4)

    @pl.when(wid == _NUM_WORKERS - 1)
    def _():
        work(_LAST_W, 6)


_sc_call = functools.partial(
    pl.kernel,
    out_type=jax.ShapeDtypeStruct((_N_ATOMS,), jnp.float32),
    mesh=plsc.VectorSubcoreMesh(core_axis_name="c", subcore_axis_name="s"),
    compiler_params=pltpu.CompilerParams(needs_layout_passes=False),
    scratch_types=[
        pltpu.VMEM((_N_SPECIES,), jnp.float32),
        pltpu.VMEM((_PER_W,), jnp.float32),
        pltpu.VMEM((_PER_W,), jnp.int32),
        pltpu.VMEM((_PER_W,), jnp.float32),
        pltpu.SemaphoreType.DMA,
    ],
)(_sc_body)


@jax.jit
def kernel(x, Z, shift):
    return _sc_call(x, Z, shift.reshape(_N_SPECIES))


# speedup vs baseline: 25.2887x; 1.0001x over previous
import functools

import jax
import jax.numpy as jnp
from jax import lax
from jax.experimental import pallas as pl
from jax.experimental.pallas import tpu as pltpu
from jax.experimental.pallas import tpu_sc as plsc

_N_ATOMS = 100000
_NUM_WORKERS = 32
_PER_W = 3136
_LAST_W = _N_ATOMS - 31 * _PER_W
_N_SPECIES = 120
_LANES = 16


def _sc_body(x_hbm, z_hbm, tab_hbm, out_hbm, tab_v, x_v, z_v, o_v, sem):
    wid = lax.axis_index("s") * 2 + lax.axis_index("c")
    base = wid * _PER_W

    def work(size, unroll):
        cp_t = pltpu.make_async_copy(tab_hbm, tab_v, sem)
        cp_x = pltpu.make_async_copy(
            x_hbm.at[pl.ds(base, size)], x_v.at[pl.ds(0, size)], sem)
        cp_z = pltpu.make_async_copy(
            z_hbm.at[pl.ds(base, size)], z_v.at[pl.ds(0, size)], sem)
        cp_t.start()
        cp_x.start()
        cp_z.start()
        cp_t.wait()
        cp_x.wait()
        cp_z.wait()

        step = unroll * _LANES

        def body(i, carry):
            off = i * step
            for j in range(unroll):
                o = off + j * _LANES
                z = z_v[pl.ds(o, _LANES)]
                s = plsc.load_gather(tab_v, [z])
                o_v[pl.ds(o, _LANES)] = x_v[pl.ds(o, _LANES)] - s
            return carry

        lax.fori_loop(0, size // step, body, 0)
        pltpu.sync_copy(o_v.at[pl.ds(0, size)], out_hbm.at[pl.ds(base, size)])

    @pl.when(wid < _NUM_WORKERS - 1)
    def _():
        work(_PER_W, 4)

    @pl.when(wid == _NUM_WORKERS - 1)
    def _():
        work(_LAST_W, 6)


_sc_call = functools.partial(
    pl.kernel,
    out_type=jax.ShapeDtypeStruct((_N_ATOMS,), jnp.float32),
    mesh=plsc.VectorSubcoreMesh(core_axis_name="c", subcore_axis_name="s"),
    compiler_params=pltpu.CompilerParams(
        needs_layout_passes=False, skip_device_barrier=True),
    scratch_types=[
        pltpu.VMEM((_N_SPECIES,), jnp.float32),
        pltpu.VMEM((_PER_W,), jnp.float32),
        pltpu.VMEM((_PER_W,), jnp.int32),
        pltpu.VMEM((_PER_W,), jnp.float32),
        pltpu.SemaphoreType.DMA,
    ],
)(_sc_body)


@jax.jit
def kernel(x, Z, shift):
    return _sc_call(x, Z, shift.reshape(_N_SPECIES))
